# baseline (speedup 1.0000x reference)
import functools

import jax
import jax.numpy as jnp
from jax import lax
from jax.experimental import pallas as pl
from jax.experimental.pallas import tpu as pltpu
from jax.experimental.pallas import tpu_sc as plsc

BATCH = 4096
MAX_LEN = 50
EMB = 32
PACK = 4
NUM_WORKERS = 32
ITEMS = BATCH // NUM_WORKERS
NBUF = 2


def _sc_dots(idx4, off, table4, wb):
    mesh = plsc.VectorSubcoreMesh(core_axis_name="c", subcore_axis_name="s")

    @functools.partial(
        pl.kernel,
        out_type=jax.ShapeDtypeStruct((BATCH,), jnp.float32),
        mesh=mesh,
        scratch_types=[
            pltpu.VMEM((MAX_LEN * ITEMS,), jnp.int32),
            pltpu.VMEM((MAX_LEN * ITEMS,), jnp.int32),
            pltpu.VMEM((MAX_LEN * EMB * 16,), jnp.float32),
            pltpu.VMEM((ITEMS,), jnp.float32),
        ]
        + [pltpu.VMEM((ITEMS, PACK * EMB), jnp.float32) for _ in range(NBUF)]
        + [pltpu.SemaphoreType.DMA for _ in range(NBUF)],
        compiler_params=pltpu.CompilerParams(needs_layout_passes=False),
    )
    def sc_kernel(idx_hbm, off_hbm, table_hbm, wb_hbm, out_hbm,
                  idx_v, off_v, wb_v, acc_v, *bufs):
        rows = bufs[:NBUF]
        sems = bufs[NBUF:]
        wid = lax.axis_index("c") * 16 + lax.axis_index("s")
        pltpu.sync_copy(wb_hbm, wb_v)
        pltpu.sync_copy(idx_hbm.at[wid], idx_v)
        pltpu.sync_copy(off_hbm.at[wid], off_v)

        def gdesc(l, b):
            return pltpu.make_async_copy(
                table_hbm.at[idx_v.at[pl.ds(l * ITEMS, ITEMS)]],
                rows[b], sems[b])

        for b in range(NBUF):
            gdesc(b, b).start()

        for g in range(ITEMS // 16):
            acc_v[pl.ds(g * 16, 16)] = jnp.zeros((16,), jnp.float32)

        iota16 = lax.iota(jnp.int32, 16)

        @pl.loop(0, MAX_LEN, step=NBUF)
        def _(base):
            for b in range(NBUF):
                l = base + b
                gdesc(l, b).wait()
                rows_b = rows[b]

                for g in range(ITEMS // 16):
                    offv = off_v[pl.ds(l * ITEMS + g * 16, 16)]
                    riota = iota16 + (g * 16)

                    def ebody(eo, acc, offv=offv, riota=riota, rows_b=rows_b,
                              l=l):
                        for u in range(4):
                            e = eo * 4 + u
                            wv = wb_v[pl.ds(l * (EMB * 16) + e * 16, 16)]
                            vals = plsc.load_gather(
                                rows_b, [riota, offv + e])
                            acc = acc + vals * wv
                        return acc

                    acc = lax.fori_loop(
                        0, EMB // 4, ebody, jnp.zeros((16,), jnp.float32))
                    plsc.addupdate(acc_v.at[pl.ds(g * 16, 16)], acc)

                @pl.when(l + NBUF < MAX_LEN)
                def _():
                    gdesc(l + NBUF, b).start()

        pltpu.sync_copy(acc_v, out_hbm.at[pl.ds(wid * ITEMS, ITEMS)])

    return sc_kernel(idx4, off, table4, wb)


def _tc_finish_body(p_ref, b_ref, o_ref):
    o_ref[...] = jax.nn.sigmoid(p_ref[...] + b_ref[0, 0])


_PACK_ROWS = 2000


def _tc_pack_body(x0, x1, x2, x3, o_ref):
    o_ref[:, 0:32] = x0[...]
    o_ref[:, 32:64] = x1[...]
    o_ref[:, 64:96] = x2[...]
    o_ref[:, 96:128] = x3[...]


def _tc_pack(table):
    v = table.shape[0]
    q = v // PACK
    nblk = q // _PACK_ROWS
    specs = [
        pl.BlockSpec((_PACK_ROWS, EMB),
                     lambda i, s=s: (i + s * nblk, 0))
        for s in range(PACK)
    ]
    return pl.pallas_call(
        _tc_pack_body,
        grid=(nblk,),
        in_specs=specs,
        out_specs=pl.BlockSpec((_PACK_ROWS, PACK * EMB), lambda i: (i, 0)),
        out_shape=jax.ShapeDtypeStruct((q, PACK * EMB), jnp.float32),
    )(table, table, table, table)


def kernel(x, table, W, b):
    xi = x.astype(jnp.int32)
    xi = jnp.transpose(xi.reshape(NUM_WORKERS, ITEMS, MAX_LEN), (0, 2, 1))
    xi = xi.reshape(NUM_WORKERS, MAX_LEN * ITEMS)
    qrows = table.shape[0] // PACK
    idx4 = xi % qrows
    off = (xi // qrows) * EMB
    table4 = _tc_pack(table)
    wb = jnp.broadcast_to(
        W.astype(jnp.float32).reshape(MAX_LEN, EMB, 1), (MAX_LEN, EMB, 16)
    ).reshape(-1)

    dots = _sc_dots(idx4, off, table4, wb)

    out = pl.pallas_call(
        _tc_finish_body,
        out_shape=jax.ShapeDtypeStruct((BATCH, 1), jnp.float32),
    )(dots.reshape(BATCH, 1), b.reshape(1, 1))
    return out


# speedup vs baseline: 1.2108x; 1.2108x over previous
import functools

import jax
import jax.numpy as jnp
from jax import lax
from jax.experimental import pallas as pl
from jax.experimental.pallas import tpu as pltpu
from jax.experimental.pallas import tpu_sc as plsc

BATCH = 4096
MAX_LEN = 50
EMB = 32
NUM_WORKERS = 32
ITEMS = BATCH // NUM_WORKERS


def _sc_partials(xw, table, w2d):
    mesh = plsc.VectorSubcoreMesh(core_axis_name="c", subcore_axis_name="s")

    NBUF = 5

    @functools.partial(
        pl.kernel,
        out_type=jax.ShapeDtypeStruct((BATCH, EMB), jnp.float32),
        mesh=mesh,
        scratch_types=[
            pltpu.VMEM((MAX_LEN, ITEMS), jnp.int32),
            pltpu.VMEM((ITEMS, EMB), jnp.float32),
            pltpu.VMEM((MAX_LEN, EMB), jnp.float32),
        ]
        + [pltpu.VMEM((ITEMS, EMB), jnp.float32) for _ in range(NBUF)]
        + [pltpu.SemaphoreType.DMA for _ in range(NBUF)],
        compiler_params=pltpu.CompilerParams(use_tc_tiling_on_sc=False),
    )
    def sc_kernel(xw_hbm, table_hbm, w_hbm, out_hbm, idx_v, acc_v, w_v, *bufs):
        rows = bufs[:NBUF]
        sems = bufs[NBUF:]
        wid = lax.axis_index("c") * 16 + lax.axis_index("s")
        pltpu.sync_copy(w_hbm, w_v)
        pltpu.sync_copy(xw_hbm.at[wid], idx_v)

        def gdesc(l, b):
            return pltpu.make_async_copy(
                table_hbm.at[idx_v.at[l]], rows[b], sems[b])

        for b in range(NBUF):
            gdesc(b, b).start()

        @pl.loop(0, ITEMS)
        def _(j):
            acc_v[j, pl.ds(0, 16)] = jnp.zeros((16,), jnp.float32)
            acc_v[j, pl.ds(16, 16)] = jnp.zeros((16,), jnp.float32)

        @pl.loop(0, MAX_LEN, step=NBUF)
        def _(base):
            for b in range(NBUF):
                l = base + b
                gdesc(l, b).wait()
                wl0 = w_v[l, pl.ds(0, 16)]
                wl1 = w_v[l, pl.ds(16, 16)]
                rows_b = rows[b]

                @pl.loop(0, ITEMS, step=4)
                def _(j):
                    for u in range(4):
                        r0 = rows_b[j + u, pl.ds(0, 16)]
                        r1 = rows_b[j + u, pl.ds(16, 16)]
                        plsc.addupdate(acc_v.at[j + u, pl.ds(0, 16)], r0 * wl0)
                        plsc.addupdate(acc_v.at[j + u, pl.ds(16, 16)], r1 * wl1)

                @pl.when(l + NBUF < MAX_LEN)
                def _():
                    gdesc(l + NBUF, b).start()

        pltpu.sync_copy(acc_v, out_hbm.at[pl.ds(wid * ITEMS, ITEMS)])

    return sc_kernel(xw, table, w2d)


def _tc_finish_body(p_ref, b_ref, o_ref):
    s = jnp.sum(p_ref[...], axis=1, keepdims=True) + b_ref[0, 0]
    o_ref[...] = jax.nn.sigmoid(s)


def kernel(x, table, W, b):
    xw = x.astype(jnp.int32).reshape(NUM_WORKERS, ITEMS, MAX_LEN)
    xw = jnp.transpose(xw, (0, 2, 1))
    w2d = W.astype(jnp.float32).reshape(MAX_LEN, EMB)

    partials = _sc_partials(xw, table, w2d)

    out = pl.pallas_call(
        _tc_finish_body,
        out_shape=jax.ShapeDtypeStruct((BATCH, 1), jnp.float32),
    )(partials, b.reshape(1, 1))
    return out
